# baseline (speedup 1.0000x reference)
import functools

import jax
import jax.numpy as jnp
from jax import lax
from jax.experimental import pallas as pl
from jax.experimental.pallas import tpu as pltpu
from jax.experimental.pallas import tpu_sc as plsc

_NB = 16
_NVB = 400


def _sc_gather(table, idx_w, n_rows):
    info = plsc.get_sparse_core_info()
    nw = info.num_cores * info.num_subcores
    per_w = idx_w.shape[1]
    n_vec = per_w // 16

    mesh = plsc.VectorSubcoreMesh(core_axis_name="c", subcore_axis_name="s")

    @functools.partial(
        pl.kernel,
        mesh=mesh,
        out_type=jax.ShapeDtypeStruct((nw, per_w * 3), jnp.float32),
        scratch_types=[
            pltpu.VMEM((table.shape[0] * 3,), jnp.float32),
            pltpu.VMEM((per_w,), jnp.int32),
            pltpu.VMEM((per_w * 3,), jnp.float32),
        ],
        compiler_params=pltpu.CompilerParams(needs_layout_passes=False),
    )
    def gather_k(table_hbm, idx_hbm, out_hbm, table_v, idx_v, rows_v):
        wid = lax.axis_index("s") * info.num_cores + lax.axis_index("c")
        pltpu.sync_copy(table_hbm, table_v)
        pltpu.sync_copy(idx_hbm.at[wid], idx_v)

        def body(i, carry):
            base = pl.multiple_of(i * 16, 16)
            iv = idx_v[pl.ds(base, 16)] * 3
            obase = pl.multiple_of(i * 48, 16)
            for c in range(3):
                vals = plsc.load_gather(table_v, [iv + c])
                rows_v[pl.ds(obase + c * 16, 16)] = vals
            return carry

        lax.fori_loop(0, n_vec, body, 0)
        pltpu.sync_copy(rows_v, out_hbm.at[wid])

    return gather_k(table.reshape(-1), idx_w)


def _conv_body(s_ref, g_ref, c_ref, o_ref):
    s = s_ref[...]
    s2 = jnp.sum(s * s, axis=0, keepdims=True)
    sn = s * (1.0 / jnp.maximum(jnp.sqrt(s2), 1e-12))
    sx, sy, sz = sn[0:1, :], sn[1:2, :], sn[2:3, :]

    g = g_ref[...]
    c = c_ref[...]
    dx = g[:, 0, :] - c[:, 0:1]
    dy = g[:, 1, :] - c[:, 1:2]
    dz = g[:, 2, :] - c[:, 2:3]
    n2 = dx * dx + dy * dy + dz * dz
    inv = 1.0 / jnp.maximum(jnp.sqrt(n2), 1e-12)
    dx, dy, dz = dx * inv, dy * inv, dz * inv
    acc = jnp.zeros((o_ref.shape[0], 256), jnp.float32)
    for n in range(_NB):
        th = (dx[:, n:n + 1] * sx + dy[:, n:n + 1] * sy
              + dz[:, n:n + 1] * sz)
        acc = jnp.maximum(acc, th)
    o_ref[...] = (acc[:, 0:64] + acc[:, 64:128]
                  + acc[:, 128:192] + acc[:, 192:256])


def _tc_conv(directions, g3, table, n_rows):
    grid = (n_rows // _NVB,)
    return pl.pallas_call(
        _conv_body,
        grid=grid,
        in_specs=[
            pl.BlockSpec((3, 256), lambda i: (0, 0)),
            pl.BlockSpec((_NVB, 3, _NB), lambda i: (i, 0, 0)),
            pl.BlockSpec((_NVB, 3), lambda i: (i, 0)),
        ],
        out_specs=pl.BlockSpec((_NVB, 64), lambda i: (i, 0)),
        out_shape=jax.ShapeDtypeStruct((n_rows, 64), jnp.float32),
        compiler_params=pltpu.CompilerParams(
            dimension_semantics=("parallel",),
        ),
    )(directions, g3, table)


def kernel(neighbor_index, vertices, directions):
    bs, v, nb = neighbor_index.shape
    n_rows = bs * v
    n_idx = bs * v * nb
    nw = 32
    table = vertices.reshape(n_rows, 3)
    idx_w = (
        neighbor_index + (jnp.arange(bs, dtype=jnp.int32) * v)[:, None, None]
    ).reshape(nw, n_idx // nw)
    gathered = _sc_gather(table, idx_w, n_idx)
    g3 = gathered.reshape(n_rows, 3, nb)
    out = _tc_conv(directions, g3, table, n_rows)
    return out.reshape(bs, v, 64)


# speedup vs baseline: 3.9112x; 3.9112x over previous
import functools

import jax
import jax.numpy as jnp
from jax import lax
from jax.experimental import pallas as pl
from jax.experimental.pallas import tpu as pltpu
from jax.experimental.pallas import tpu_sc as plsc

_NB = 16
_NVB = 160


def _sc_gather(table, idx_w, n_rows):
    info = plsc.get_sparse_core_info()
    nw = info.num_cores * info.num_subcores
    per_w = idx_w.shape[1]
    n_vec = per_w // 16

    mesh = plsc.VectorSubcoreMesh(core_axis_name="c", subcore_axis_name="s")

    @functools.partial(
        pl.kernel,
        mesh=mesh,
        out_type=jax.ShapeDtypeStruct((nw, per_w * 3), jnp.float32),
        scratch_types=[
            pltpu.VMEM((table.shape[0] * 3,), jnp.float32),
            pltpu.VMEM((per_w,), jnp.int32),
            pltpu.VMEM((per_w * 3,), jnp.float32),
        ],
        compiler_params=pltpu.CompilerParams(needs_layout_passes=False),
    )
    def gather_k(table_hbm, idx_hbm, out_hbm, table_v, idx_v, rows_v):
        wid = lax.axis_index("s") * info.num_cores + lax.axis_index("c")
        pltpu.sync_copy(table_hbm, table_v)
        pltpu.sync_copy(idx_hbm.at[wid], idx_v)

        def body(i, carry):
            base = pl.multiple_of(i * 16, 16)
            iv = idx_v[pl.ds(base, 16)] * 3
            obase = pl.multiple_of(i * 48, 16)
            for c in range(3):
                vals = plsc.load_gather(table_v, [iv + c])
                rows_v[pl.ds(obase + c * 16, 16)] = vals
            return carry

        lax.fori_loop(0, n_vec, body, 0)
        pltpu.sync_copy(rows_v, out_hbm.at[wid])

    return gather_k(table.reshape(-1), idx_w)


def _conv_body(s_ref, g_ref, c_ref, o_ref):
    s = s_ref[...]
    s2 = jnp.sum(s * s, axis=0, keepdims=True)
    sn = s * (1.0 / jnp.maximum(jnp.sqrt(s2), 1e-12))
    sx, sy, sz = sn[0:1, :], sn[1:2, :], sn[2:3, :]

    g = g_ref[...]
    c = c_ref[...]
    dx = g[:, 0, :] - c[:, 0:1]
    dy = g[:, 1, :] - c[:, 1:2]
    dz = g[:, 2, :] - c[:, 2:3]
    n2 = dx * dx + dy * dy + dz * dz
    inv = 1.0 / jnp.maximum(jnp.sqrt(n2), 1e-12)
    dx, dy, dz = dx * inv, dy * inv, dz * inv
    th = (dx[:, :, None] * sx.reshape(1, 1, 256)
          + dy[:, :, None] * sy.reshape(1, 1, 256)
          + dz[:, :, None] * sz.reshape(1, 1, 256))
    th = jnp.maximum(th, 0.0)
    m8 = jnp.maximum(th[:, 0:8, :], th[:, 8:16, :])
    m = jnp.max(m8, axis=1)
    o_ref[...] = (m[:, 0:64] + m[:, 64:128]
                  + m[:, 128:192] + m[:, 192:256])


def _tc_conv(directions, g3, table, n_rows):
    grid = (n_rows // _NVB,)
    return pl.pallas_call(
        _conv_body,
        grid=grid,
        in_specs=[
            pl.BlockSpec((3, 256), lambda i: (0, 0)),
            pl.BlockSpec((_NVB, 3, _NB), lambda i: (i, 0, 0)),
            pl.BlockSpec((_NVB, 3), lambda i: (i, 0)),
        ],
        out_specs=pl.BlockSpec((_NVB, 64), lambda i: (i, 0)),
        out_shape=jax.ShapeDtypeStruct((n_rows, 64), jnp.float32),
        compiler_params=pltpu.CompilerParams(
            dimension_semantics=("parallel",),
        ),
    )(directions, g3, table)


def kernel(neighbor_index, vertices, directions):
    bs, v, nb = neighbor_index.shape
    n_rows = bs * v
    n_idx = bs * v * nb
    nw = 32
    table = vertices.reshape(n_rows, 3)
    idx_w = (
        neighbor_index + (jnp.arange(bs, dtype=jnp.int32) * v)[:, None, None]
    ).reshape(nw, n_idx // nw)
    gathered = _sc_gather(table, idx_w, n_idx)
    g3 = gathered.reshape(n_rows, 3, nb)
    out = _tc_conv(directions, g3, table, n_rows)
    return out.reshape(bs, v, 64)


# speedup vs baseline: 4.7900x; 1.2247x over previous
import functools

import jax
import jax.numpy as jnp
from jax import lax
from jax.experimental import pallas as pl
from jax.experimental.pallas import tpu as pltpu
from jax.experimental.pallas import tpu_sc as plsc

_NB = 16
_NVB = 160


def _sc_gather_diff(table, idx_w):
    info = plsc.get_sparse_core_info()
    nw = info.num_cores * info.num_subcores
    per_w = idx_w.shape[1]
    n_vec = per_w // 16

    mesh = plsc.VectorSubcoreMesh(core_axis_name="c", subcore_axis_name="s")

    @functools.partial(
        pl.kernel,
        mesh=mesh,
        out_type=jax.ShapeDtypeStruct((nw, 3 * per_w), jnp.float32),
        scratch_types=[
            pltpu.VMEM(table.shape, jnp.float32),
            pltpu.VMEM((per_w,), jnp.int32),
            pltpu.VMEM((3 * per_w,), jnp.float32),
        ],
        compiler_params=pltpu.CompilerParams(needs_layout_passes=False),
    )
    def gather_k(table_hbm, idx_hbm, out_hbm, table_v, idx_v, rows_v):
        wid = lax.axis_index("s") * info.num_cores + lax.axis_index("c")
        pltpu.sync_copy(table_hbm, table_v)
        pltpu.sync_copy(idx_hbm.at[wid], idx_v)

        def body(i, carry):
            base = pl.multiple_of(i * 16, 16)
            iv = idx_v[pl.ds(base, 16)] * 3
            rbase = (wid * n_vec + i) * 3
            for c in range(3):
                vals = plsc.load_gather(table_v, [iv + c])
                cen = plsc.load_gather(
                    table_v, [jnp.full((16,), rbase + c, jnp.int32)]
                )
                rows_v[pl.ds(c * per_w + base, 16)] = vals - cen
            return carry

        lax.fori_loop(0, n_vec, body, 0)
        pltpu.sync_copy(rows_v, out_hbm.at[wid])

    return gather_k(table, idx_w)


def _conv_body(s_ref, g_ref, o_ref):
    s = s_ref[...]
    s2 = jnp.sum(s * s, axis=0, keepdims=True)
    sn = s * (1.0 / jnp.maximum(jnp.sqrt(s2), 1e-12))
    sx = sn[0:1, :].reshape(1, 1, 256)
    sy = sn[1:2, :].reshape(1, 1, 256)
    sz = sn[2:3, :].reshape(1, 1, 256)

    g = g_ref[...]
    dx, dy, dz = g[0], g[1], g[2]
    n2 = dx * dx + dy * dy + dz * dz
    inv = lax.rsqrt(jnp.maximum(n2, 1e-24))
    dx, dy, dz = dx * inv, dy * inv, dz * inv
    th = (dx[:, :, None] * sx + dy[:, :, None] * sy
          + dz[:, :, None] * sz)
    m8 = jnp.maximum(th[:, 0:8, :], th[:, 8:16, :])
    m = jnp.max(m8, axis=1)
    m = jnp.maximum(m, 0.0)
    mm = m[:, 0:128] + m[:, 128:256]
    o_ref[...] = mm[:, 0:64] + mm[:, 64:128]


def _tc_conv(directions, g3, n_rows):
    grid = (n_rows // _NVB,)
    return pl.pallas_call(
        _conv_body,
        grid=grid,
        in_specs=[
            pl.BlockSpec((3, 256), lambda i: (0, 0)),
            pl.BlockSpec((3, _NVB, _NB), lambda i: (0, i, 0)),
        ],
        out_specs=pl.BlockSpec((_NVB, 64), lambda i: (i, 0)),
        out_shape=jax.ShapeDtypeStruct((n_rows, 64), jnp.float32),
        compiler_params=pltpu.CompilerParams(
            dimension_semantics=("parallel",),
        ),
    )(directions, g3)


def kernel(neighbor_index, vertices, directions):
    bs, v, nb = neighbor_index.shape
    n_rows = bs * v
    n_idx = bs * v * nb
    nw = 32
    table = vertices.reshape(n_rows * 3)
    idx_w = (
        neighbor_index + (jnp.arange(bs, dtype=jnp.int32) * v)[:, None, None]
    ).reshape(nw, n_idx // nw)
    diffs = _sc_gather_diff(table, idx_w)
    g3 = (
        diffs.reshape(nw, 3, n_idx // nw)
        .transpose(1, 0, 2)
        .reshape(3, n_rows, nb)
    )
    out = _tc_conv(directions, g3, n_rows)
    return out.reshape(bs, v, 64)


# speedup vs baseline: 5.3769x; 1.1225x over previous
import functools

import jax
import jax.numpy as jnp
from jax import lax
from jax.experimental import pallas as pl
from jax.experimental.pallas import tpu as pltpu
from jax.experimental.pallas import tpu_sc as plsc

_NB = 16
_NVB = 160


def _sc_gather_diff(table, idx_w):
    info = plsc.get_sparse_core_info()
    nw = info.num_cores * info.num_subcores
    per_w = idx_w.shape[1]
    n_vec = per_w // 16

    mesh = plsc.VectorSubcoreMesh(core_axis_name="c", subcore_axis_name="s")

    @functools.partial(
        pl.kernel,
        mesh=mesh,
        out_type=[jax.ShapeDtypeStruct((nw, per_w), jnp.float32)] * 3,
        scratch_types=[
            pltpu.VMEM(table.shape, jnp.float32),
            pltpu.VMEM((per_w,), jnp.int32),
            pltpu.VMEM((per_w,), jnp.float32),
            pltpu.VMEM((per_w,), jnp.float32),
            pltpu.VMEM((per_w,), jnp.float32),
        ],
        compiler_params=pltpu.CompilerParams(needs_layout_passes=False),
    )
    def gather_k(table_hbm, idx_hbm, ox_hbm, oy_hbm, oz_hbm, table_v, idx_v,
                 rvx, rvy, rvz):
        wid = lax.axis_index("s") * info.num_cores + lax.axis_index("c")
        pltpu.sync_copy(table_hbm, table_v)
        pltpu.sync_copy(idx_hbm.at[wid], idx_v)
        planes = (rvx, rvy, rvz)

        def body(i, carry):
            base = pl.multiple_of(i * 16, 16)
            iv = idx_v[pl.ds(base, 16)] * 3
            rbase = (wid * n_vec + i) * 3
            for c in range(3):
                vals = plsc.load_gather(table_v, [iv + c])
                cen = plsc.load_gather(
                    table_v, [jnp.full((16,), rbase + c, jnp.int32)]
                )
                planes[c][pl.ds(base, 16)] = vals - cen
            return carry

        lax.fori_loop(0, n_vec, body, 0)
        pltpu.sync_copy(rvx, ox_hbm.at[wid])
        pltpu.sync_copy(rvy, oy_hbm.at[wid])
        pltpu.sync_copy(rvz, oz_hbm.at[wid])

    return gather_k(table, idx_w)


def _conv_body(s_ref, x_ref, y_ref, z_ref, o_ref):
    s = s_ref[...]
    s2 = jnp.sum(s * s, axis=0, keepdims=True)
    sn = s * (1.0 / jnp.maximum(jnp.sqrt(s2), 1e-12))
    sx = sn[0:1, :].reshape(1, 1, 256)
    sy = sn[1:2, :].reshape(1, 1, 256)
    sz = sn[2:3, :].reshape(1, 1, 256)

    dx, dy, dz = x_ref[...], y_ref[...], z_ref[...]
    n2 = dx * dx + dy * dy + dz * dz
    inv = lax.rsqrt(jnp.maximum(n2, 1e-24))
    bf = jnp.bfloat16
    dx = (dx * inv).astype(bf)
    dy = (dy * inv).astype(bf)
    dz = (dz * inv).astype(bf)
    sx, sy, sz = sx.astype(bf), sy.astype(bf), sz.astype(bf)
    th = (dx[:, :, None] * sx + dy[:, :, None] * sy
          + dz[:, :, None] * sz)
    m8 = jnp.maximum(th[:, 0:8, :], th[:, 8:16, :])
    m = jnp.max(m8, axis=1).astype(jnp.float32)
    m = jnp.maximum(m, 0.0)
    mm = m[:, 0:128] + m[:, 128:256]
    o_ref[...] = mm[:, 0:64] + mm[:, 64:128]


def _tc_conv(directions, gx, gy, gz, n_rows):
    grid = (n_rows // _NVB,)
    plane = pl.BlockSpec((_NVB, _NB), lambda i: (i, 0))
    return pl.pallas_call(
        _conv_body,
        grid=grid,
        in_specs=[
            pl.BlockSpec((3, 256), lambda i: (0, 0)),
            plane, plane, plane,
        ],
        out_specs=pl.BlockSpec((_NVB, 64), lambda i: (i, 0)),
        out_shape=jax.ShapeDtypeStruct((n_rows, 64), jnp.float32),
        compiler_params=pltpu.CompilerParams(
            dimension_semantics=("parallel",),
        ),
    )(directions, gx, gy, gz)


def kernel(neighbor_index, vertices, directions):
    bs, v, nb = neighbor_index.shape
    n_rows = bs * v
    n_idx = bs * v * nb
    nw = 32
    table = vertices.reshape(n_rows * 3)
    idx_w = (
        neighbor_index + (jnp.arange(bs, dtype=jnp.int32) * v)[:, None, None]
    ).reshape(nw, n_idx // nw)
    ox, oy, oz = _sc_gather_diff(table, idx_w)
    gx = ox.reshape(n_rows, nb)
    gy = oy.reshape(n_rows, nb)
    gz = oz.reshape(n_rows, nb)
    out = _tc_conv(directions, gx, gy, gz, n_rows)
    return out.reshape(bs, v, 64)


# speedup vs baseline: 5.5948x; 1.0405x over previous
import functools

import jax
import jax.numpy as jnp
from jax import lax
from jax.experimental import pallas as pl
from jax.experimental.pallas import tpu as pltpu
from jax.experimental.pallas import tpu_sc as plsc

_NB = 16
_NVB = 400


def _sc_gather_diff(table, idx_w):
    info = plsc.get_sparse_core_info()
    nw = info.num_cores * info.num_subcores
    per_w = idx_w.shape[1]
    n_vec = per_w // 16

    mesh = plsc.VectorSubcoreMesh(core_axis_name="c", subcore_axis_name="s")

    @functools.partial(
        pl.kernel,
        mesh=mesh,
        out_type=[jax.ShapeDtypeStruct((nw, per_w), jnp.float32)] * 3,
        scratch_types=[
            pltpu.VMEM(table.shape, jnp.float32),
            pltpu.VMEM((per_w,), jnp.int32),
            pltpu.VMEM((per_w,), jnp.float32),
            pltpu.VMEM((per_w,), jnp.float32),
            pltpu.VMEM((per_w,), jnp.float32),
        ],
        compiler_params=pltpu.CompilerParams(needs_layout_passes=False),
    )
    def gather_k(table_hbm, idx_hbm, ox_hbm, oy_hbm, oz_hbm, table_v, idx_v,
                 rvx, rvy, rvz):
        wid = lax.axis_index("s") * info.num_cores + lax.axis_index("c")
        pltpu.sync_copy(table_hbm, table_v)
        pltpu.sync_copy(idx_hbm.at[wid], idx_v)
        planes = (rvx, rvy, rvz)

        def body(i, carry):
            base = pl.multiple_of(i * 16, 16)
            iv = idx_v[pl.ds(base, 16)] * 3
            rbase = (wid * n_vec + i) * 3
            for c in range(3):
                vals = plsc.load_gather(table_v, [iv + c])
                cen = plsc.load_gather(
                    table_v, [jnp.full((16,), rbase + c, jnp.int32)]
                )
                planes[c][pl.ds(base, 16)] = vals - cen
            return carry

        lax.fori_loop(0, n_vec, body, 0)
        pltpu.sync_copy(rvx, ox_hbm.at[wid])
        pltpu.sync_copy(rvy, oy_hbm.at[wid])
        pltpu.sync_copy(rvz, oz_hbm.at[wid])

    return gather_k(table, idx_w)


def _conv_body(s_ref, x_ref, y_ref, z_ref, o_ref):
    s = s_ref[...]
    s2 = jnp.sum(s * s, axis=0, keepdims=True)
    sn = s * (1.0 / jnp.maximum(jnp.sqrt(s2), 1e-12))
    sx = sn[0:1, :].reshape(1, 1, 256)
    sy = sn[1:2, :].reshape(1, 1, 256)
    sz = sn[2:3, :].reshape(1, 1, 256)

    dx, dy, dz = x_ref[...], y_ref[...], z_ref[...]
    n2 = dx * dx + dy * dy + dz * dz
    inv = lax.rsqrt(jnp.maximum(n2, 1e-24))
    bf = jnp.bfloat16
    dx = (dx * inv).astype(bf)
    dy = (dy * inv).astype(bf)
    dz = (dz * inv).astype(bf)
    sx, sy, sz = sx.astype(bf), sy.astype(bf), sz.astype(bf)
    th = (dx[:, :, None] * sx + dy[:, :, None] * sy
          + dz[:, :, None] * sz)
    m8 = jnp.maximum(th[:, 0:8, :], th[:, 8:16, :])
    m = jnp.max(m8, axis=1).astype(jnp.float32)
    m = jnp.maximum(m, 0.0)
    mm = m[:, 0:128] + m[:, 128:256]
    o_ref[...] = mm[:, 0:64] + mm[:, 64:128]


def _tc_conv(directions, gx, gy, gz, n_rows):
    grid = (n_rows // _NVB,)
    plane = pl.BlockSpec((_NVB, _NB), lambda i: (i, 0))
    return pl.pallas_call(
        _conv_body,
        grid=grid,
        in_specs=[
            pl.BlockSpec((3, 256), lambda i: (0, 0)),
            plane, plane, plane,
        ],
        out_specs=pl.BlockSpec((_NVB, 64), lambda i: (i, 0)),
        out_shape=jax.ShapeDtypeStruct((n_rows, 64), jnp.float32),
        compiler_params=pltpu.CompilerParams(
            dimension_semantics=("parallel",),
        ),
    )(directions, gx, gy, gz)


def kernel(neighbor_index, vertices, directions):
    bs, v, nb = neighbor_index.shape
    n_rows = bs * v
    n_idx = bs * v * nb
    nw = 32
    table = vertices.reshape(n_rows * 3)
    idx_w = (
        neighbor_index + (jnp.arange(bs, dtype=jnp.int32) * v)[:, None, None]
    ).reshape(nw, n_idx // nw)
    ox, oy, oz = _sc_gather_diff(table, idx_w)
    gx = ox.reshape(n_rows, nb)
    gy = oy.reshape(n_rows, nb)
    gz = oz.reshape(n_rows, nb)
    out = _tc_conv(directions, gx, gy, gz, n_rows)
    return out.reshape(bs, v, 64)


# speedup vs baseline: 5.6393x; 1.0079x over previous
import functools

import jax
import jax.numpy as jnp
from jax import lax
from jax.experimental import pallas as pl
from jax.experimental.pallas import tpu as pltpu
from jax.experimental.pallas import tpu_sc as plsc

_NB = 16
_NVB = 1000


def _sc_gather_diff(table, idx_w):
    info = plsc.get_sparse_core_info()
    nw = info.num_cores * info.num_subcores
    per_w = idx_w.shape[1]
    n_vec = per_w // 16

    mesh = plsc.VectorSubcoreMesh(core_axis_name="c", subcore_axis_name="s")

    @functools.partial(
        pl.kernel,
        mesh=mesh,
        out_type=[jax.ShapeDtypeStruct((nw, per_w), jnp.float32)] * 3,
        scratch_types=[
            pltpu.VMEM(table.shape, jnp.float32),
            pltpu.VMEM((per_w,), jnp.int32),
            pltpu.VMEM((per_w,), jnp.float32),
            pltpu.VMEM((per_w,), jnp.float32),
            pltpu.VMEM((per_w,), jnp.float32),
        ],
        compiler_params=pltpu.CompilerParams(needs_layout_passes=False),
    )
    def gather_k(table_hbm, idx_hbm, ox_hbm, oy_hbm, oz_hbm, table_v, idx_v,
                 rvx, rvy, rvz):
        wid = lax.axis_index("s") * info.num_cores + lax.axis_index("c")
        pltpu.sync_copy(table_hbm, table_v)
        pltpu.sync_copy(idx_hbm.at[wid], idx_v)
        planes = (rvx, rvy, rvz)

        def body(i, carry):
            base = pl.multiple_of(i * 16, 16)
            iv = idx_v[pl.ds(base, 16)] * 3
            rbase = (wid * n_vec + i) * 3
            for c in range(3):
                vals = plsc.load_gather(table_v, [iv + c])
                cen = plsc.load_gather(
                    table_v, [jnp.full((16,), rbase + c, jnp.int32)]
                )
                planes[c][pl.ds(base, 16)] = vals - cen
            return carry

        lax.fori_loop(0, n_vec, body, 0)
        pltpu.sync_copy(rvx, ox_hbm.at[wid])
        pltpu.sync_copy(rvy, oy_hbm.at[wid])
        pltpu.sync_copy(rvz, oz_hbm.at[wid])

    return gather_k(table, idx_w)


def _conv_body(s_ref, x_ref, y_ref, z_ref, o_ref):
    s = s_ref[...]
    s2 = jnp.sum(s * s, axis=0, keepdims=True)
    sn = s * (1.0 / jnp.maximum(jnp.sqrt(s2), 1e-12))
    sx = sn[0:1, :].reshape(1, 1, 256)
    sy = sn[1:2, :].reshape(1, 1, 256)
    sz = sn[2:3, :].reshape(1, 1, 256)

    dx, dy, dz = x_ref[...], y_ref[...], z_ref[...]
    n2 = dx * dx + dy * dy + dz * dz
    inv = lax.rsqrt(jnp.maximum(n2, 1e-24))
    bf = jnp.bfloat16
    dx = (dx * inv).astype(bf)
    dy = (dy * inv).astype(bf)
    dz = (dz * inv).astype(bf)
    sx, sy, sz = sx.astype(bf), sy.astype(bf), sz.astype(bf)
    th = (dx[:, :, None] * sx + dy[:, :, None] * sy
          + dz[:, :, None] * sz)
    m8 = jnp.maximum(th[:, 0:8, :], th[:, 8:16, :])
    m = jnp.max(m8, axis=1).astype(jnp.float32)
    m = jnp.maximum(m, 0.0)
    mm = m[:, 0:128] + m[:, 128:256]
    o_ref[...] = mm[:, 0:64] + mm[:, 64:128]


def _tc_conv(directions, gx, gy, gz, n_rows):
    grid = (n_rows // _NVB,)
    plane = pl.BlockSpec((_NVB, _NB), lambda i: (i, 0))
    return pl.pallas_call(
        _conv_body,
        grid=grid,
        in_specs=[
            pl.BlockSpec((3, 256), lambda i: (0, 0)),
            plane, plane, plane,
        ],
        out_specs=pl.BlockSpec((_NVB, 64), lambda i: (i, 0)),
        out_shape=jax.ShapeDtypeStruct((n_rows, 64), jnp.float32),
        compiler_params=pltpu.CompilerParams(
            dimension_semantics=("parallel",),
        ),
    )(directions, gx, gy, gz)


def kernel(neighbor_index, vertices, directions):
    bs, v, nb = neighbor_index.shape
    n_rows = bs * v
    n_idx = bs * v * nb
    nw = 32
    table = vertices.reshape(n_rows * 3)
    idx_w = (
        neighbor_index + (jnp.arange(bs, dtype=jnp.int32) * v)[:, None, None]
    ).reshape(nw, n_idx // nw)
    ox, oy, oz = _sc_gather_diff(table, idx_w)
    gx = ox.reshape(n_rows, nb)
    gy = oy.reshape(n_rows, nb)
    gz = oz.reshape(n_rows, nb)
    out = _tc_conv(directions, gx, gy, gz, n_rows)
    return out.reshape(bs, v, 64)


# speedup vs baseline: 6.2961x; 1.1165x over previous
import functools

import jax
import jax.numpy as jnp
from jax import lax
from jax.experimental import pallas as pl
from jax.experimental.pallas import tpu as pltpu
from jax.experimental.pallas import tpu_sc as plsc

_NB = 16
_NVB = 1000


def _sc_gather_diff(table, idx_w):
    info = plsc.get_sparse_core_info()
    nw = info.num_cores * info.num_subcores
    per_w = idx_w.shape[1]
    n_vec = per_w // 16

    mesh = plsc.VectorSubcoreMesh(core_axis_name="c", subcore_axis_name="s")

    @functools.partial(
        pl.kernel,
        mesh=mesh,
        out_type=[jax.ShapeDtypeStruct((nw, per_w), jnp.float32)] * 3,
        scratch_types=[
            pltpu.VMEM(table.shape, jnp.float32),
            pltpu.VMEM((per_w,), jnp.int32),
            pltpu.VMEM((per_w,), jnp.float32),
            pltpu.VMEM((per_w,), jnp.float32),
            pltpu.VMEM((per_w,), jnp.float32),
        ],
        compiler_params=pltpu.CompilerParams(needs_layout_passes=False),
    )
    def gather_k(table_hbm, idx_hbm, ox_hbm, oy_hbm, oz_hbm, table_v, idx_v,
                 rvx, rvy, rvz):
        wid = lax.axis_index("s") * info.num_cores + lax.axis_index("c")
        pltpu.sync_copy(table_hbm, table_v)
        pltpu.sync_copy(idx_hbm.at[wid], idx_v)
        planes = (rvx, rvy, rvz)

        def body(i, carry):
            base = pl.multiple_of(i * 16, 16)
            iv = idx_v[pl.ds(base, 16)] * 3
            rbase = (wid * n_vec + i) * 3
            for c in range(3):
                vals = plsc.load_gather(table_v, [iv + c])
                cen = plsc.load_gather(
                    table_v, [jnp.full((16,), rbase + c, jnp.int32)]
                )
                planes[c][pl.ds(base, 16)] = vals - cen
            return carry

        lax.fori_loop(0, n_vec, body, 0)
        pltpu.sync_copy(rvx, ox_hbm.at[wid])
        pltpu.sync_copy(rvy, oy_hbm.at[wid])
        pltpu.sync_copy(rvz, oz_hbm.at[wid])

    return gather_k(table, idx_w)


def _conv_body(s_ref, x_ref, y_ref, z_ref, o_ref):
    s = s_ref[...]
    s2 = jnp.sum(s * s, axis=0, keepdims=True)
    sn = s * (1.0 / jnp.maximum(jnp.sqrt(s2), 1e-12))
    sx = sn[0:1, :].reshape(1, 1, 256)
    sy = sn[1:2, :].reshape(1, 1, 256)
    sz = sn[2:3, :].reshape(1, 1, 256)

    dx, dy, dz = x_ref[...], y_ref[...], z_ref[...]
    n2 = dx * dx + dy * dy + dz * dz
    inv = lax.rsqrt(jnp.maximum(n2, 1e-24))
    bf = jnp.bfloat16
    dx = (dx * inv).astype(bf)
    dy = (dy * inv).astype(bf)
    dz = (dz * inv).astype(bf)
    sx, sy, sz = sx.astype(bf), sy.astype(bf), sz.astype(bf)
    th = (dx[:, :, None] * sx + dy[:, :, None] * sy
          + dz[:, :, None] * sz)
    m8 = jnp.maximum(th[:, 0:8, :], th[:, 8:16, :])
    m4 = jnp.maximum(m8[:, 0:4, :], m8[:, 4:8, :])
    m2 = jnp.maximum(m4[:, 0:2, :], m4[:, 2:4, :])
    m1 = jnp.maximum(m2[:, 0:1, :], m2[:, 1:2, :])
    m = m1.reshape(m1.shape[0], 256).astype(jnp.float32)
    m = jnp.maximum(m, 0.0)
    mm = m[:, 0:128] + m[:, 128:256]
    o_ref[...] = mm[:, 0:64] + mm[:, 64:128]


def _tc_conv(directions, gx, gy, gz, n_rows):
    grid = (n_rows // _NVB,)
    plane = pl.BlockSpec((_NVB, _NB), lambda i: (i, 0))
    return pl.pallas_call(
        _conv_body,
        grid=grid,
        in_specs=[
            pl.BlockSpec((3, 256), lambda i: (0, 0)),
            plane, plane, plane,
        ],
        out_specs=pl.BlockSpec((_NVB, 64), lambda i: (i, 0)),
        out_shape=jax.ShapeDtypeStruct((n_rows, 64), jnp.float32),
        compiler_params=pltpu.CompilerParams(
            dimension_semantics=("parallel",),
        ),
    )(directions, gx, gy, gz)


def kernel(neighbor_index, vertices, directions):
    bs, v, nb = neighbor_index.shape
    n_rows = bs * v
    n_idx = bs * v * nb
    nw = 32
    table = vertices.reshape(n_rows * 3)
    idx_w = (
        neighbor_index + (jnp.arange(bs, dtype=jnp.int32) * v)[:, None, None]
    ).reshape(nw, n_idx // nw)
    ox, oy, oz = _sc_gather_diff(table, idx_w)
    gx = ox.reshape(n_rows, nb)
    gy = oy.reshape(n_rows, nb)
    gz = oz.reshape(n_rows, nb)
    out = _tc_conv(directions, gx, gy, gz, n_rows)
    return out.reshape(bs, v, 64)
